# baseline (speedup 1.0000x reference)
import jax
import jax.numpy as jnp
from jax.experimental import pallas as pl
from jax.experimental.pallas import tpu as pltpu
from jax.experimental.pallas import tpu_sc as plsc

B, C_IN, C_OUT, N, M, K = 8, 64, 64, 8192, 512, 8
BLK_N = 2048
GATHER_WINDOW = 128


def _sc_gather_rows(rows, idx):
    n_idx = idx.shape[1]

    @pl.kernel(
        out_type=jax.ShapeDtypeStruct((n_idx, rows.shape[1]), rows.dtype),
        mesh=plsc.VectorSubcoreMesh(core_axis_name="core",
                                    subcore_axis_name="subcore"))
    def gather_kernel(x_hbm, i_hbm, o_hbm):
        def body(i_vmem, o_vmem):
            pltpu.sync_copy(x_hbm.at[i_vmem.at[0]], o_vmem)

        pltpu.emit_pipeline(
            body,
            grid=(n_idx // GATHER_WINDOW,),
            in_specs=[pl.BlockSpec((1, GATHER_WINDOW),
                                   index_map=lambda i: (0, i))],
            out_specs=[pl.BlockSpec((GATHER_WINDOW, rows.shape[1]),
                                    index_map=lambda i: (i, 0))],
            core_axis_name="subcore",
            dimension_semantics=(pltpu.PARALLEL,),
        )(i_hbm, o_hbm)

    return gather_kernel(rows, idx)


def _transpose_kernel(x_ref, o_ref):
    o_ref[0] = jnp.swapaxes(x_ref[0], 0, 1)


def _transpose_bcn_to_bnc(x):
    return pl.pallas_call(
        _transpose_kernel,
        grid=(B, N // BLK_N),
        in_specs=[pl.BlockSpec((1, C_IN, BLK_N), lambda bb, ii: (bb, 0, ii))],
        out_specs=pl.BlockSpec((1, BLK_N, C_IN), lambda bb, ii: (bb, ii, 0)),
        out_shape=jax.ShapeDtypeStruct((B, N, C_IN), jnp.float32),
        compiler_params=pltpu.CompilerParams(
            dimension_semantics=("parallel", "parallel")),
    )(x)


def _knn_conv_kernel(x_ref, y_ref, z_ref, par_ref, wt_ref, bias_ref, out_ref,
                     u_ref, yn_ref, sim_ref, acc_ref):
    i = pl.program_id(1)

    @pl.when(i == 0)
    def _prep():
        yv = y_ref[0]
        norm = jnp.sqrt(jnp.sum(yv * yv, axis=0, keepdims=True))
        yn_ref[...] = (yv / jnp.clip(norm, 1e-12, None)).astype(jnp.bfloat16)
        zpair = z_ref[0]
        zv = jnp.where(par_ref[...] > 0, zpair[:, C_IN:], zpair[:, :C_IN])
        zv = zv.astype(jnp.bfloat16)
        for k in range(1, K):
            u = jax.lax.dot_general(
                wt_ref[k].astype(jnp.bfloat16), zv, (((1,), (1,)), ((), ())),
                preferred_element_type=jnp.float32)
            uhi = u.astype(jnp.bfloat16)
            u_ref[k - 1, :C_OUT] = uhi
            u_ref[k - 1, C_OUT:] = (u - uhi.astype(jnp.float32)).astype(jnp.bfloat16)

    xb = x_ref[0]
    xnorm = jnp.sqrt(jnp.sum(xb * xb, axis=0, keepdims=True))
    xn = (xb / jnp.clip(xnorm, 1e-12, None)).astype(jnp.bfloat16)
    sim_ref[...] = jax.lax.dot_general(
        yn_ref[...], xn, (((0,), (0,)), ((), ())),
        preferred_element_type=jnp.float32)
    acc_ref[...] = jax.lax.dot_general(
        wt_ref[0].astype(jnp.bfloat16), xb.astype(jnp.bfloat16),
        (((1,), (0,)), ((), ())),
        preferred_element_type=jnp.float32) + bias_ref[...]

    for k in range(K - 1):
        sim = sim_ref[...]
        mx = jnp.max(sim, axis=0, keepdims=True)
        onehot0 = sim == mx
        oh = onehot0.astype(jnp.bfloat16)
        maxcnt = jnp.max(jnp.sum(oh, axis=0).astype(jnp.float32))

        @pl.when(maxcnt == jnp.float32(1))
        def _fast():
            hilo = jax.lax.dot_general(
                u_ref[k], oh, (((1,), (0,)), ((), ())),
                preferred_element_type=jnp.float32)
            acc_ref[...] += hilo[:C_OUT] + hilo[C_OUT:]
            sim_ref[...] = jnp.where(onehot0, -1e30, sim)

        @pl.when(maxcnt != jnp.float32(1))
        def _tied():
            iota_m = jax.lax.broadcasted_iota(
                jnp.int32, (M, BLK_N), 0).astype(jnp.float32)
            t = jnp.where(onehot0, iota_m, jnp.float32(M))
            cmin = jnp.min(t, axis=0, keepdims=True)
            onehot = t == cmin
            hilo = jax.lax.dot_general(
                u_ref[k], onehot.astype(jnp.bfloat16),
                (((1,), (0,)), ((), ())),
                preferred_element_type=jnp.float32)
            acc_ref[...] += hilo[:C_OUT] + hilo[C_OUT:]
            sim_ref[...] = jnp.where(onehot, -1e30, sim)

    out_ref[0] = acc_ref[...]


@jax.jit
def kernel(x, y, indices, W, b):
    xt_pairs = jnp.swapaxes(x, 1, 2).reshape(B * N // 2, 2 * C_IN)
    idx32 = indices.astype(jnp.int32)
    flat_idx = (idx32[None, :] +
                (jnp.arange(B, dtype=jnp.int32) * N)[:, None])
    pair_idx = (flat_idx // 2).reshape(1, B * M)
    parity = (idx32 % 2).astype(jnp.float32).reshape(M, 1)
    zt = _sc_gather_rows(xt_pairs, pair_idx).reshape(B, M, 2 * C_IN)
    wt = jnp.transpose(W, (2, 0, 1))
    bias = b.reshape(C_OUT, 1)

    grid = (B, N // BLK_N)
    out = pl.pallas_call(
        _knn_conv_kernel,
        grid=grid,
        in_specs=[
            pl.BlockSpec((1, C_IN, BLK_N), lambda bb, ii: (bb, 0, ii)),
            pl.BlockSpec((1, C_IN, M), lambda bb, ii: (bb, 0, 0)),
            pl.BlockSpec((1, M, 2 * C_IN), lambda bb, ii: (bb, 0, 0)),
            pl.BlockSpec((M, 1), lambda bb, ii: (0, 0)),
            pl.BlockSpec((K, C_OUT, C_IN), lambda bb, ii: (0, 0, 0)),
            pl.BlockSpec((C_OUT, 1), lambda bb, ii: (0, 0)),
        ],
        out_specs=pl.BlockSpec((1, C_OUT, BLK_N), lambda bb, ii: (bb, 0, ii)),
        out_shape=jax.ShapeDtypeStruct((B, C_OUT, N), jnp.float32),
        scratch_shapes=[
            pltpu.VMEM((K - 1, 2 * C_OUT, M), jnp.bfloat16),
            pltpu.VMEM((C_IN, M), jnp.bfloat16),
            pltpu.VMEM((M, BLK_N), jnp.float32),
            pltpu.VMEM((C_OUT, BLK_N), jnp.float32),
        ],
        compiler_params=pltpu.CompilerParams(
            dimension_semantics=("parallel", "arbitrary")),
    )(x, y, zt, parity, wt, bias)
    return out


# speedup vs baseline: 1.5153x; 1.5153x over previous
import jax
import jax.numpy as jnp
from jax.experimental import pallas as pl
from jax.experimental.pallas import tpu as pltpu
from jax.experimental.pallas import tpu_sc as plsc

B, C_IN, C_OUT, N, M, K = 8, 64, 64, 8192, 512, 8
BLK_N = 2048
GATHER_WINDOW = 128


def _sc_gather_rows(rows, idx):
    n_idx = idx.shape[1]

    @pl.kernel(
        out_type=jax.ShapeDtypeStruct((n_idx, rows.shape[1]), rows.dtype),
        mesh=plsc.VectorSubcoreMesh(core_axis_name="core",
                                    subcore_axis_name="subcore"))
    def gather_kernel(x_hbm, i_hbm, o_hbm):
        def body(i_vmem, o_vmem):
            pltpu.sync_copy(x_hbm.at[i_vmem.at[0]], o_vmem)

        pltpu.emit_pipeline(
            body,
            grid=(n_idx // GATHER_WINDOW,),
            in_specs=[pl.BlockSpec((1, GATHER_WINDOW),
                                   index_map=lambda i: (0, i))],
            out_specs=[pl.BlockSpec((GATHER_WINDOW, rows.shape[1]),
                                    index_map=lambda i: (i, 0))],
            core_axis_name="subcore",
            dimension_semantics=(pltpu.PARALLEL,),
        )(i_hbm, o_hbm)

    return gather_kernel(rows, idx)


def _knn_conv_kernel(x_ref, y_ref, z_ref, par_ref, wt_ref, bias_ref, out_ref,
                     u_ref, yn_ref):
    i = pl.program_id(1)

    @pl.when(i == 0)
    def _prep():
        yv = y_ref[0]
        norm = jnp.sqrt(jnp.sum(yv * yv, axis=0, keepdims=True))
        yn_ref[...] = (yv / jnp.clip(norm, 1e-12, None)).astype(jnp.bfloat16)
        zpair = z_ref[0]
        zv = jnp.where(par_ref[...] > 0, zpair[:, C_IN:], zpair[:, :C_IN])
        zv = zv.astype(jnp.bfloat16)
        for k in range(1, K):
            u = jax.lax.dot_general(
                wt_ref[k].astype(jnp.bfloat16), zv, (((1,), (1,)), ((), ())),
                preferred_element_type=jnp.float32)
            uhi = u.astype(jnp.bfloat16)
            u_ref[k - 1, :C_OUT] = uhi
            u_ref[k - 1, C_OUT:] = (u - uhi.astype(jnp.float32)).astype(jnp.bfloat16)

    xb = x_ref[0]
    xnorm = jnp.sqrt(jnp.sum(xb * xb, axis=0, keepdims=True))
    xn = (xb / jnp.clip(xnorm, 1e-12, None)).astype(jnp.bfloat16)
    sim = jax.lax.dot_general(
        yn_ref[...], xn, (((0,), (0,)), ((), ())),
        preferred_element_type=jnp.float32)
    acc = jax.lax.dot_general(
        wt_ref[0].astype(jnp.bfloat16), xb.astype(jnp.bfloat16),
        (((1,), (0,)), ((), ())),
        preferred_element_type=jnp.float32) + bias_ref[...]

    iota_m = jax.lax.broadcasted_iota(
        jnp.int32, (M, BLK_N), 0).astype(jnp.float32)
    fm = jnp.float32(M)
    for k in range(K - 1):
        mx = jnp.max(sim, axis=0, keepdims=True)
        t = jnp.where(sim == mx, iota_m, fm)
        cmin = jnp.min(t, axis=0, keepdims=True)
        onehot = t == cmin
        oh = onehot.astype(jnp.bfloat16)
        hilo = jax.lax.dot_general(
            u_ref[k], oh, (((1,), (0,)), ((), ())),
            preferred_element_type=jnp.float32)
        acc = acc + hilo[:C_OUT] + hilo[C_OUT:]
        sim = jnp.where(onehot, -1e30, sim)

    out_ref[0] = acc


@jax.jit
def kernel(x, y, indices, W, b):
    xt_pairs = jnp.swapaxes(x, 1, 2).reshape(B * N // 2, 2 * C_IN)
    idx32 = indices.astype(jnp.int32)
    flat_idx = (idx32[None, :] +
                (jnp.arange(B, dtype=jnp.int32) * N)[:, None])
    pair_idx = (flat_idx // 2).reshape(1, B * M)
    parity = (idx32 % 2).astype(jnp.float32).reshape(M, 1)
    zt = _sc_gather_rows(xt_pairs, pair_idx).reshape(B, M, 2 * C_IN)
    wt = jnp.transpose(W, (2, 0, 1))
    bias = b.reshape(C_OUT, 1)

    grid = (B, N // BLK_N)
    out = pl.pallas_call(
        _knn_conv_kernel,
        grid=grid,
        in_specs=[
            pl.BlockSpec((1, C_IN, BLK_N), lambda bb, ii: (bb, 0, ii)),
            pl.BlockSpec((1, C_IN, M), lambda bb, ii: (bb, 0, 0)),
            pl.BlockSpec((1, M, 2 * C_IN), lambda bb, ii: (bb, 0, 0)),
            pl.BlockSpec((M, 1), lambda bb, ii: (0, 0)),
            pl.BlockSpec((K, C_OUT, C_IN), lambda bb, ii: (0, 0, 0)),
            pl.BlockSpec((C_OUT, 1), lambda bb, ii: (0, 0)),
        ],
        out_specs=pl.BlockSpec((1, C_OUT, BLK_N), lambda bb, ii: (bb, 0, ii)),
        out_shape=jax.ShapeDtypeStruct((B, C_OUT, N), jnp.float32),
        scratch_shapes=[
            pltpu.VMEM((K - 1, 2 * C_OUT, M), jnp.bfloat16),
            pltpu.VMEM((C_IN, M), jnp.bfloat16),
        ],
        compiler_params=pltpu.CompilerParams(
            dimension_semantics=("parallel", "arbitrary")),
    )(x, y, zt, parity, wt, bias)
    return out


# speedup vs baseline: 1.5193x; 1.0026x over previous
import jax
import jax.numpy as jnp
from jax.experimental import pallas as pl
from jax.experimental.pallas import tpu as pltpu
from jax.experimental.pallas import tpu_sc as plsc

B, C_IN, C_OUT, N, M, K = 8, 64, 64, 8192, 512, 8
BLK_N = 2048
GATHER_WINDOW = 128


def _sc_gather_rows(rows, idx):
    n_idx = idx.shape[1]

    @pl.kernel(
        out_type=jax.ShapeDtypeStruct((n_idx, rows.shape[1]), rows.dtype),
        mesh=plsc.VectorSubcoreMesh(core_axis_name="core",
                                    subcore_axis_name="subcore"))
    def gather_kernel(x_hbm, i_hbm, o_hbm):
        def body(i_vmem, o_vmem):
            pltpu.sync_copy(x_hbm.at[i_vmem.at[0]], o_vmem)

        pltpu.emit_pipeline(
            body,
            grid=(n_idx // GATHER_WINDOW,),
            in_specs=[pl.BlockSpec((1, GATHER_WINDOW),
                                   index_map=lambda i: (0, i))],
            out_specs=[pl.BlockSpec((GATHER_WINDOW, rows.shape[1]),
                                    index_map=lambda i: (i, 0))],
            core_axis_name="subcore",
            dimension_semantics=(pltpu.PARALLEL,),
        )(i_hbm, o_hbm)

    return gather_kernel(rows, idx)


def _knn_conv_kernel(x_ref, y_ref, z_ref, par_ref, wt_ref, bias_ref, out_ref,
                     u_ref, yn_ref):
    i = pl.program_id(1)

    @pl.when(i == 0)
    def _prep():
        yv = y_ref[0]
        norm = jnp.sqrt(jnp.sum(yv * yv, axis=0, keepdims=True))
        yn_ref[...] = (yv / jnp.clip(norm, 1e-12, None)).astype(jnp.bfloat16)
        zpair = z_ref[0]
        zv = jnp.where(par_ref[...] > 0, zpair[:, C_IN:], zpair[:, :C_IN])
        zv = zv.astype(jnp.bfloat16)
        for k in range(1, K):
            u = jax.lax.dot_general(
                wt_ref[k].astype(jnp.bfloat16), zv, (((1,), (1,)), ((), ())),
                preferred_element_type=jnp.float32)
            uhi = u.astype(jnp.bfloat16)
            u_ref[k - 1, :C_OUT] = uhi
            u_ref[k - 1, C_OUT:] = (u - uhi.astype(jnp.float32)).astype(jnp.bfloat16)

    xb = x_ref[0]
    xnorm = jnp.sqrt(jnp.sum(xb * xb, axis=0, keepdims=True))
    xn = (xb / jnp.clip(xnorm, 1e-12, None)).astype(jnp.bfloat16)
    sim = jax.lax.dot_general(
        yn_ref[...], xn, (((0,), (0,)), ((), ())),
        preferred_element_type=jnp.float32)
    acc = jax.lax.dot_general(
        wt_ref[0].astype(jnp.bfloat16), xb.astype(jnp.bfloat16),
        (((1,), (0,)), ((), ())),
        preferred_element_type=jnp.float32) + bias_ref[...]

    iota_m = jax.lax.broadcasted_iota(
        jnp.int32, (M, BLK_N), 0).astype(jnp.float32)
    fm = jnp.float32(M)
    for k in range(K - 1):
        mx = jnp.max(sim, axis=0, keepdims=True)
        t = jnp.where(sim == mx, iota_m, fm)
        cmin = jnp.min(t, axis=0, keepdims=True)
        onehot = t == cmin
        oh = onehot.astype(jnp.bfloat16)
        hilo = jax.lax.dot_general(
            u_ref[k], oh, (((1,), (0,)), ((), ())),
            preferred_element_type=jnp.float32)
        acc = acc + hilo[:C_OUT] + hilo[C_OUT:]
        if k < K - 2:
            sim = jnp.where(onehot, -1e30, sim)

    out_ref[0] = acc


@jax.jit
def kernel(x, y, indices, W, b):
    xt_pairs = jnp.swapaxes(x, 1, 2).reshape(B * N // 2, 2 * C_IN)
    idx32 = indices.astype(jnp.int32)
    flat_idx = (idx32[None, :] +
                (jnp.arange(B, dtype=jnp.int32) * N)[:, None])
    pair_idx = (flat_idx // 2).reshape(1, B * M)
    parity = (idx32 % 2).astype(jnp.float32).reshape(M, 1)
    zt = _sc_gather_rows(xt_pairs, pair_idx).reshape(B, M, 2 * C_IN)
    wt = jnp.transpose(W, (2, 0, 1))
    bias = b.reshape(C_OUT, 1)

    grid = (B, N // BLK_N)
    out = pl.pallas_call(
        _knn_conv_kernel,
        grid=grid,
        in_specs=[
            pl.BlockSpec((1, C_IN, BLK_N), lambda bb, ii: (bb, 0, ii)),
            pl.BlockSpec((1, C_IN, M), lambda bb, ii: (bb, 0, 0)),
            pl.BlockSpec((1, M, 2 * C_IN), lambda bb, ii: (bb, 0, 0)),
            pl.BlockSpec((M, 1), lambda bb, ii: (0, 0)),
            pl.BlockSpec((K, C_OUT, C_IN), lambda bb, ii: (0, 0, 0)),
            pl.BlockSpec((C_OUT, 1), lambda bb, ii: (0, 0)),
        ],
        out_specs=pl.BlockSpec((1, C_OUT, BLK_N), lambda bb, ii: (bb, 0, ii)),
        out_shape=jax.ShapeDtypeStruct((B, C_OUT, N), jnp.float32),
        scratch_shapes=[
            pltpu.VMEM((K - 1, 2 * C_OUT, M), jnp.bfloat16),
            pltpu.VMEM((C_IN, M), jnp.bfloat16),
        ],
        compiler_params=pltpu.CompilerParams(
            dimension_semantics=("parallel", "arbitrary")),
    )(x, y, zt, parity, wt, bias)
    return out
